# baseline (speedup 1.0000x reference)
import functools

import jax
import jax.numpy as jnp
from jax import lax
from jax.experimental import pallas as pl
from jax.experimental.pallas import tpu as pltpu
from jax.experimental.pallas import tpu_sc as plsc

N = 10000
NP = 10240
E = 320000
D = 128
NC = 2
NS = 16
NW = NC * NS
EPW = E // NW
C = 80
NBUF = 4
NBATCH = 32
EPP = NBATCH * NBUF * C
NCHUNK = EPP // C
GROUPS = C // 16
RPT = NP // NS


def _sc_agg_body(h_hbm, idx_hbm, z128_hbm, acc_out, *rest):
    idx_v = list(rest[0:2 * NBUF])
    rows_v = list(rest[2 * NBUF:3 * NBUF])
    acc_sh = rest[3 * NBUF]
    isem = list(rest[3 * NBUF + 1:5 * NBUF + 1])
    gsem = list(rest[5 * NBUF + 1:6 * NBUF + 1])
    ssem = list(rest[6 * NBUF + 1:7 * NBUF + 1])
    c = lax.axis_index("c")
    s = lax.axis_index("s")
    wid = s * NC + c
    cbase = wid * NCHUNK

    def issue_idx(j, h, b):
        pltpu.async_copy(idx_hbm.at[cbase + j * NBUF + b],
                         idx_v[h * NBUF + b], isem[h * NBUF + b])

    def wait_idx(h, b):
        pltpu.make_async_copy(idx_hbm.at[0], idx_v[h * NBUF + b],
                              isem[h * NBUF + b]).wait()

    def issue_gather(h, b):
        pltpu.async_copy(h_hbm.at[idx_v[h * NBUF + b].at[0]], rows_v[b],
                         gsem[b])

    def wait_gather(b):
        pltpu.make_async_copy(h_hbm.at[pl.ds(0, C)], rows_v[b],
                              gsem[b]).wait()

    def issue_scatter(h, b):
        pltpu.async_copy(rows_v[b], acc_sh.at[idx_v[h * NBUF + b].at[1]],
                         ssem[b], add=True)

    def wait_scatter(b):
        pltpu.make_async_copy(h_hbm.at[pl.ds(0, C)], rows_v[b],
                              ssem[b]).wait()

    pltpu.sync_copy(z128_hbm.at[pl.ds(s * RPT, RPT)],
                    acc_sh.at[pl.ds(s * RPT, RPT)])
    plsc.subcore_barrier()

    for b in range(NBUF):
        issue_idx(0, 0, b)
    for b in range(NBUF):
        wait_idx(0, b)
        issue_gather(0, b)
    for b in range(NBUF):
        issue_idx(1, 1, b)
    for b in range(NBUF):
        wait_gather(b)
        issue_scatter(0, b)

    def _pair(i, carry):
        for (off, h) in ((1, 1), (2, 0)):
            j = 2 * i + off
            for b in range(NBUF):
                wait_scatter(b)
                wait_idx(h, b)
                issue_gather(h, b)
            for b in range(NBUF):
                issue_idx(j + 1, 1 - h, b)
            for b in range(NBUF):
                wait_gather(b)
                issue_scatter(h, b)
        return carry

    lax.fori_loop(0, (NBATCH - 2) // 2, _pair, 0)

    for b in range(NBUF):
        wait_scatter(b)
        wait_idx(1, b)
        issue_gather(1, b)
    for b in range(NBUF):
        wait_gather(b)
        issue_scatter(1, b)
    for b in range(NBUF):
        wait_scatter(b)
    plsc.subcore_barrier()

    row = c * NP + s * RPT
    pltpu.sync_copy(acc_sh.at[pl.ds(s * RPT, RPT)],
                    acc_out.at[pl.ds(row, RPT)])


def _sc_cnt_body(eye_hbm, idx_hbm, ea_hbm, z128_hbm, cnt_out, *rest):
    idx_v = list(rest[0:NBUF])
    ea_v = list(rest[NBUF:2 * NBUF])
    k_v = list(rest[2 * NBUF:3 * NBUF])
    oneh_v = list(rest[3 * NBUF:4 * NBUF])
    cnt_sh = rest[4 * NBUF]
    isem = list(rest[4 * NBUF + 1:5 * NBUF + 1])
    esem = list(rest[5 * NBUF + 1:6 * NBUF + 1])
    gsem = list(rest[6 * NBUF + 1:7 * NBUF + 1])
    ssem = list(rest[7 * NBUF + 1:8 * NBUF + 1])
    c = lax.axis_index("c")
    s = lax.axis_index("s")
    wid = s * NC + c
    cbase = wid * NCHUNK

    def issue_idx(t, b):
        pltpu.async_copy(idx_hbm.at[cbase + t], idx_v[b], isem[b])
        pltpu.async_copy(ea_hbm.at[cbase + t], ea_v[b], esem[b])

    def wait_idx(b):
        pltpu.make_async_copy(idx_hbm.at[0], idx_v[b], isem[b]).wait()
        pltpu.make_async_copy(ea_hbm.at[0], ea_v[b], esem[b]).wait()

    def issue_gather(b):
        def _grp(g, cc):
            k_v[b][pl.ds(g * 16, 16)] = (
                ea_v[b][0, pl.ds(g * 16, 16)] * 3
                + ea_v[b][1, pl.ds(g * 16, 16)]
                + (wid * NBUF + b) * 16)
            return cc
        lax.fori_loop(0, GROUPS, _grp, 0)
        pltpu.async_copy(eye_hbm.at[k_v[b]], oneh_v[b], gsem[b])

    def wait_gather(b):
        pltpu.make_async_copy(z128_hbm.at[pl.ds(0, C)], oneh_v[b],
                              gsem[b]).wait()

    def issue_scatter(b):
        pltpu.async_copy(oneh_v[b], cnt_sh.at[idx_v[b].at[1]], ssem[b],
                         add=True)

    def wait_scatter(b):
        pltpu.make_async_copy(z128_hbm.at[pl.ds(0, C)], oneh_v[b],
                              ssem[b]).wait()

    pltpu.sync_copy(z128_hbm.at[pl.ds(s * RPT, RPT)],
                    cnt_sh.at[pl.ds(s * RPT, RPT)])
    plsc.subcore_barrier()

    for b in range(NBUF):
        issue_idx(b, b)

    def _batch(i, carry):
        t4 = i * NBUF
        for b in range(NBUF):
            wait_idx(b)
            issue_gather(b)
        for b in range(NBUF):
            wait_gather(b)
            issue_scatter(b)
        for b in range(NBUF):
            wait_scatter(b)
            issue_idx(t4 + NBUF + b, b)
        return carry

    lax.fori_loop(0, NCHUNK // NBUF - 1, _batch, 0)

    for b in range(NBUF):
        wait_idx(b)
        issue_gather(b)
    for b in range(NBUF):
        wait_gather(b)
        issue_scatter(b)
    for b in range(NBUF):
        wait_scatter(b)
    plsc.subcore_barrier()

    row = c * NP + s * RPT
    pltpu.sync_copy(cnt_sh.at[pl.ds(s * RPT, RPT)],
                    cnt_out.at[pl.ds(row, RPT)])


_SC_CACHE = {}


def _sc_agg(*args):
    if "agg" not in _SC_CACHE:
        _SC_CACHE["agg"] = functools.partial(
            pl.kernel,
            out_type=jax.ShapeDtypeStruct((NC * NP, D), jnp.float32),
            mesh=plsc.VectorSubcoreMesh(core_axis_name="c",
                                        subcore_axis_name="s"),
            scratch_types=(
                [pltpu.VMEM((2, C), jnp.int32) for _ in range(2 * NBUF)]
                + [pltpu.VMEM((C, D), jnp.float32) for _ in range(NBUF)]
                + [pltpu.VMEM_SHARED((NP, D), jnp.float32)]
                + [pltpu.SemaphoreType.DMA for _ in range(4 * NBUF)]
            ),
        )(_sc_agg_body)
    return _SC_CACHE["agg"](*args)


def _sc_cnt(*args):
    if "cnt" not in _SC_CACHE:
        _SC_CACHE["cnt"] = functools.partial(
            pl.kernel,
            out_type=jax.ShapeDtypeStruct((NC * NP, D), jnp.float32),
            mesh=plsc.VectorSubcoreMesh(core_axis_name="c",
                                        subcore_axis_name="s"),
            scratch_types=(
                [pltpu.VMEM((2, C), jnp.int32) for _ in range(NBUF)]
                + [pltpu.VMEM((2, C), jnp.int32) for _ in range(NBUF)]
                + [pltpu.VMEM((C,), jnp.int32) for _ in range(NBUF)]
                + [pltpu.VMEM((C, D), jnp.float32) for _ in range(NBUF)]
                + [pltpu.VMEM_SHARED((NP, D), jnp.float32)]
                + [pltpu.SemaphoreType.DMA for _ in range(4 * NBUF)]
            ),
        )(_sc_cnt_body)
    return _SC_CACHE["cnt"](*args)


def _tc_embed_body(x_ref, e1_ref, e2_ref, out_ref):
    kx = x_ref[:, 0] * 3 + x_ref[:, 1]
    onehot = (kx[:, None] == lax.broadcasted_iota(jnp.int32, (1, 16), 1)
              ).astype(jnp.float32)
    rows = [e1_ref[k // 3] + e2_ref[k % 3] for k in range(9)]
    combo = jnp.stack(rows + [jnp.zeros((D,), jnp.float32)] * 7)
    out_ref[...] = jnp.dot(onehot, combo,
                           precision=lax.Precision.HIGHEST,
                           preferred_element_type=jnp.float32)


_tc_embed = pl.pallas_call(
    _tc_embed_body,
    out_shape=jax.ShapeDtypeStruct((N, D), jnp.float32),
)


def _tc_update_body(relu_out, acc_ref, cnt_ref, hprev_ref, e1_ref, e2_ref,
                    w1_ref, b1_ref, w2_ref, b2_ref, g_ref, bt_ref, out_ref):
    accsum = acc_ref[0:N] + acc_ref[NP:NP + N]
    cntsum = cnt_ref[0:N] + cnt_ref[NP:NP + N]
    rows = [e1_ref[k // 3] + e2_ref[k % 3] for k in range(9)]
    combo = jnp.stack(rows + [jnp.zeros((D,), jnp.float32)] * (D - 9))
    slconst = e1_ref[4] + e2_ref[0]
    agg = (accsum + hprev_ref[...] + slconst[None, :]
           + jnp.dot(cntsum, combo, precision=lax.Precision.HIGHEST,
                     preferred_element_type=jnp.float32))
    hid = jnp.maximum(
        jnp.dot(agg, w1_ref[...], preferred_element_type=jnp.float32)
        + b1_ref[...][None, :], 0.0)
    h2 = (jnp.dot(hid, w2_ref[...], preferred_element_type=jnp.float32)
          + b2_ref[...][None, :])
    mu = jnp.mean(h2, axis=0, keepdims=True)
    var = jnp.mean((h2 - mu) ** 2, axis=0, keepdims=True)
    out = (h2 - mu) * lax.rsqrt(var + 1e-5) * g_ref[...][None, :] \
        + bt_ref[...][None, :]
    if relu_out:
        out = jnp.maximum(out, 0.0)
    out_ref[...] = out


def _tc_update(relu_out):
    return pl.pallas_call(
        functools.partial(_tc_update_body, relu_out),
        out_shape=jax.ShapeDtypeStruct((N, D), jnp.float32),
    )


def kernel(x, edge_index, edge_attr, params):
    xi = x.astype(jnp.int32)
    src = edge_index[0].astype(jnp.int32)
    dst = edge_index[1].astype(jnp.int32)
    ea0 = edge_attr[:, 0].astype(jnp.int32)
    ea1 = edge_attr[:, 1].astype(jnp.int32)
    z128 = jnp.zeros((NP, D), jnp.float32)
    eye16 = jnp.tile(jnp.eye(16, D, dtype=jnp.float32), (NW * NBUF, 1))

    def padw(a, fill):
        a = a.reshape(NW, EPW)
        pad = jnp.full((NW, EPP - EPW), fill, a.dtype)
        return jnp.concatenate([a, pad], axis=1).reshape(NW, NCHUNK, C)

    srcp, dstp = padw(src, 0), padw(dst, N)
    idx = jnp.stack([srcp, dstp], axis=2).reshape(NW * NCHUNK, 2, C)
    ea = jnp.stack([padw(ea0, 0), padw(ea1, 0)],
                   axis=2).reshape(NW * NCHUNK, 2, C)
    cnt = _sc_cnt(eye16, idx, ea, z128)
    h = _tc_embed(xi, params['x_emb1'], params['x_emb2'])
    n_layers = len(params['layers'])
    for i, p in enumerate(params['layers']):
        acc = _sc_agg(h, idx, z128)
        h = _tc_update(i < n_layers - 1)(
            acc, cnt, h, p['edge_emb1'], p['edge_emb2'],
            p['W1'], p['b1'], p['W2'], p['b2'], p['gamma'], p['beta'])
    return h


# speedup vs baseline: 2.0037x; 2.0037x over previous
import functools

import jax
import jax.numpy as jnp
from jax import lax
from jax.experimental import pallas as pl
from jax.experimental.pallas import tpu as pltpu
from jax.experimental.pallas import tpu_sc as plsc

N = 10000
NP = 10240
E = 320000
D = 128
NC = 2
NS = 16
NW = NC * NS
EPW = E // NW
C = 80
NBUF = 4
NCHUNK = EPW // C
GROUPS = C // 16
RPT = NP // NS


def _sc_agg_body(h_hbm, idx_hbm, z128_hbm, acc_out, *rest):
    idx_v = list(rest[0:NBUF])
    rows_v = list(rest[NBUF:2 * NBUF])
    acc_sh = rest[2 * NBUF]
    isem = list(rest[2 * NBUF + 1:2 * NBUF + 1 + NBUF])
    gsem = list(rest[2 * NBUF + 1 + NBUF:2 * NBUF + 1 + 2 * NBUF])
    ssem = list(rest[2 * NBUF + 1 + 2 * NBUF:2 * NBUF + 1 + 3 * NBUF])
    c = lax.axis_index("c")
    s = lax.axis_index("s")
    wid = s * NC + c
    cbase = wid * NCHUNK

    def issue_idx(t, b):
        pltpu.async_copy(idx_hbm.at[cbase + t], idx_v[b], isem[b])

    def wait_idx(b):
        pltpu.make_async_copy(idx_hbm.at[0], idx_v[b], isem[b]).wait()

    def issue_gather(b):
        pltpu.async_copy(h_hbm.at[idx_v[b].at[0]], rows_v[b], gsem[b])

    def wait_gather(b):
        pltpu.make_async_copy(h_hbm.at[pl.ds(0, C)], rows_v[b],
                              gsem[b]).wait()

    def issue_scatter(b):
        pltpu.async_copy(rows_v[b], acc_sh.at[idx_v[b].at[1]], ssem[b],
                         add=True)

    def wait_scatter(b):
        pltpu.make_async_copy(h_hbm.at[pl.ds(0, C)], rows_v[b],
                              ssem[b]).wait()

    pltpu.sync_copy(z128_hbm.at[pl.ds(s * RPT, RPT)],
                    acc_sh.at[pl.ds(s * RPT, RPT)])
    plsc.subcore_barrier()

    for b in range(NBUF):
        issue_idx(b, b)

    def _batch(i, carry):
        t4 = i * NBUF
        for b in range(NBUF):
            wait_idx(b)
            issue_gather(b)
        for b in range(NBUF):
            wait_gather(b)
            issue_scatter(b)
        for b in range(NBUF):
            wait_scatter(b)
            issue_idx(t4 + NBUF + b, b)
        return carry

    lax.fori_loop(0, (NCHUNK - 1) // NBUF - 1, _batch, 0)

    for b in range(NBUF):
        wait_idx(b)
        issue_gather(b)
    for b in range(NBUF):
        wait_gather(b)
        issue_scatter(b)
    for b in range(NBUF):
        wait_scatter(b)

    issue_idx(NCHUNK - 1, 0)
    wait_idx(0)
    issue_gather(0)
    wait_gather(0)
    issue_scatter(0)
    wait_scatter(0)
    plsc.subcore_barrier()

    row = c * NP + s * RPT
    pltpu.sync_copy(acc_sh.at[pl.ds(s * RPT, RPT)],
                    acc_out.at[pl.ds(row, RPT)])


def _sc_cnt_body(eye_hbm, idx_hbm, ea_hbm, z128_hbm, cnt_out, *rest):
    idx_v = list(rest[0:NBUF])
    ea_v = list(rest[NBUF:2 * NBUF])
    k_v = list(rest[2 * NBUF:3 * NBUF])
    oneh_v = list(rest[3 * NBUF:4 * NBUF])
    cnt_sh = rest[4 * NBUF]
    isem = list(rest[4 * NBUF + 1:5 * NBUF + 1])
    esem = list(rest[5 * NBUF + 1:6 * NBUF + 1])
    gsem = list(rest[6 * NBUF + 1:7 * NBUF + 1])
    ssem = list(rest[7 * NBUF + 1:8 * NBUF + 1])
    c = lax.axis_index("c")
    s = lax.axis_index("s")
    wid = s * NC + c
    cbase = wid * NCHUNK

    def issue_idx(t, b):
        pltpu.async_copy(idx_hbm.at[cbase + t], idx_v[b], isem[b])
        pltpu.async_copy(ea_hbm.at[cbase + t], ea_v[b], esem[b])

    def wait_idx(b):
        pltpu.make_async_copy(idx_hbm.at[0], idx_v[b], isem[b]).wait()
        pltpu.make_async_copy(ea_hbm.at[0], ea_v[b], esem[b]).wait()

    def issue_gather(b):
        def _grp(g, cc):
            k_v[b][pl.ds(g * 16, 16)] = (
                ea_v[b][0, pl.ds(g * 16, 16)] * 3
                + ea_v[b][1, pl.ds(g * 16, 16)]
                + (wid * NBUF + b) * 16)
            return cc
        lax.fori_loop(0, GROUPS, _grp, 0)
        pltpu.async_copy(eye_hbm.at[k_v[b]], oneh_v[b], gsem[b])

    def wait_gather(b):
        pltpu.make_async_copy(z128_hbm.at[pl.ds(0, C)], oneh_v[b],
                              gsem[b]).wait()

    def issue_scatter(b):
        pltpu.async_copy(oneh_v[b], cnt_sh.at[idx_v[b].at[1]], ssem[b],
                         add=True)

    def wait_scatter(b):
        pltpu.make_async_copy(z128_hbm.at[pl.ds(0, C)], oneh_v[b],
                              ssem[b]).wait()

    pltpu.sync_copy(z128_hbm.at[pl.ds(s * RPT, RPT)],
                    cnt_sh.at[pl.ds(s * RPT, RPT)])
    plsc.subcore_barrier()

    for b in range(NBUF):
        issue_idx(b, b)

    def _batch(i, carry):
        t4 = i * NBUF
        for b in range(NBUF):
            wait_idx(b)
            issue_gather(b)
        for b in range(NBUF):
            wait_gather(b)
            issue_scatter(b)
        for b in range(NBUF):
            wait_scatter(b)
            issue_idx(t4 + NBUF + b, b)
        return carry

    lax.fori_loop(0, (NCHUNK - 1) // NBUF - 1, _batch, 0)

    for b in range(NBUF):
        wait_idx(b)
        issue_gather(b)
    for b in range(NBUF):
        wait_gather(b)
        issue_scatter(b)
    for b in range(NBUF):
        wait_scatter(b)

    issue_idx(NCHUNK - 1, 0)
    wait_idx(0)
    issue_gather(0)
    wait_gather(0)
    issue_scatter(0)
    wait_scatter(0)
    plsc.subcore_barrier()

    row = c * NP + s * RPT
    pltpu.sync_copy(cnt_sh.at[pl.ds(s * RPT, RPT)],
                    cnt_out.at[pl.ds(row, RPT)])


_SC_CACHE = {}


def _sc_agg(*args):
    if "agg" not in _SC_CACHE:
        _SC_CACHE["agg"] = functools.partial(
            pl.kernel,
            out_type=jax.ShapeDtypeStruct((NC * NP, D), jnp.float32),
            mesh=plsc.VectorSubcoreMesh(core_axis_name="c",
                                        subcore_axis_name="s"),
            scratch_types=(
                [pltpu.VMEM((2, C), jnp.int32) for _ in range(NBUF)]
                + [pltpu.VMEM((C, D), jnp.float32) for _ in range(NBUF)]
                + [pltpu.VMEM_SHARED((NP, D), jnp.float32)]
                + [pltpu.SemaphoreType.DMA for _ in range(3 * NBUF)]
            ),
        )(_sc_agg_body)
    return _SC_CACHE["agg"](*args)


def _sc_cnt(*args):
    if "cnt" not in _SC_CACHE:
        _SC_CACHE["cnt"] = functools.partial(
            pl.kernel,
            out_type=jax.ShapeDtypeStruct((NC * NP, D), jnp.float32),
            mesh=plsc.VectorSubcoreMesh(core_axis_name="c",
                                        subcore_axis_name="s"),
            scratch_types=(
                [pltpu.VMEM((2, C), jnp.int32) for _ in range(NBUF)]
                + [pltpu.VMEM((2, C), jnp.int32) for _ in range(NBUF)]
                + [pltpu.VMEM((C,), jnp.int32) for _ in range(NBUF)]
                + [pltpu.VMEM((C, D), jnp.float32) for _ in range(NBUF)]
                + [pltpu.VMEM_SHARED((NP, D), jnp.float32)]
                + [pltpu.SemaphoreType.DMA for _ in range(4 * NBUF)]
            ),
        )(_sc_cnt_body)
    return _SC_CACHE["cnt"](*args)


def _tc_embed_body(x_ref, e1_ref, e2_ref, out_ref):
    kx = x_ref[:, 0] * 3 + x_ref[:, 1]
    onehot = (kx[:, None] == lax.broadcasted_iota(jnp.int32, (1, 16), 1)
              ).astype(jnp.float32)
    rows = [e1_ref[k // 3] + e2_ref[k % 3] for k in range(9)]
    combo = jnp.stack(rows + [jnp.zeros((D,), jnp.float32)] * 7)
    out_ref[...] = jnp.dot(onehot, combo,
                           precision=lax.Precision.HIGHEST,
                           preferred_element_type=jnp.float32)


_tc_embed = pl.pallas_call(
    _tc_embed_body,
    out_shape=jax.ShapeDtypeStruct((N, D), jnp.float32),
)


def _tc_update_body(relu_out, acc_ref, cnt_ref, hprev_ref, e1_ref, e2_ref,
                    w1_ref, b1_ref, w2_ref, b2_ref, g_ref, bt_ref, out_ref):
    accsum = acc_ref[0:N] + acc_ref[NP:NP + N]
    cntsum = cnt_ref[0:N] + cnt_ref[NP:NP + N]
    rows = [e1_ref[k // 3] + e2_ref[k % 3] for k in range(9)]
    combo = jnp.stack(rows + [jnp.zeros((D,), jnp.float32)] * (D - 9))
    slconst = e1_ref[4] + e2_ref[0]
    agg = (accsum + hprev_ref[...] + slconst[None, :]
           + jnp.dot(cntsum, combo, precision=lax.Precision.HIGHEST,
                     preferred_element_type=jnp.float32))
    hid = jnp.maximum(
        jnp.dot(agg, w1_ref[...], preferred_element_type=jnp.float32)
        + b1_ref[...][None, :], 0.0)
    h2 = (jnp.dot(hid, w2_ref[...], preferred_element_type=jnp.float32)
          + b2_ref[...][None, :])
    mu = jnp.mean(h2, axis=0, keepdims=True)
    var = jnp.mean((h2 - mu) ** 2, axis=0, keepdims=True)
    out = (h2 - mu) * lax.rsqrt(var + 1e-5) * g_ref[...][None, :] \
        + bt_ref[...][None, :]
    if relu_out:
        out = jnp.maximum(out, 0.0)
    out_ref[...] = out


def _tc_update(relu_out):
    return pl.pallas_call(
        functools.partial(_tc_update_body, relu_out),
        out_shape=jax.ShapeDtypeStruct((N, D), jnp.float32),
    )


def kernel(x, edge_index, edge_attr, params):
    xi = x.astype(jnp.int32)
    src = edge_index[0].astype(jnp.int32)
    dst = edge_index[1].astype(jnp.int32)
    ea0 = edge_attr[:, 0].astype(jnp.int32)
    ea1 = edge_attr[:, 1].astype(jnp.int32)
    z128 = jnp.zeros((NP, D), jnp.float32)
    eye16 = jnp.tile(jnp.eye(16, D, dtype=jnp.float32), (NW * NBUF, 1))
    idx = jnp.stack([src.reshape(NW, NCHUNK, C),
                     dst.reshape(NW, NCHUNK, C)],
                    axis=2).reshape(NW * NCHUNK, 2, C)

    ea = jnp.stack([ea0.reshape(NW, NCHUNK, C),
                    ea1.reshape(NW, NCHUNK, C)],
                   axis=2).reshape(NW * NCHUNK, 2, C)
    cnt = _sc_cnt(eye16, idx, ea, z128)
    h = _tc_embed(xi, params['x_emb1'], params['x_emb2'])
    n_layers = len(params['layers'])
    for i, p in enumerate(params['layers']):
        acc = _sc_agg(h, idx, z128)
        h = _tc_update(i < n_layers - 1)(
            acc, cnt, h, p['edge_emb1'], p['edge_emb2'],
            p['W1'], p['b1'], p['W2'], p['b2'], p['gamma'], p['beta'])
    return h
